# Initial kernel scaffold; baseline (speedup 1.0000x reference)
import jax
import jax.numpy as jnp
from jax.experimental import pallas as pl


def kernel(inputs, n2e_in, n2e_out, x_idx, y_idx, W_n11, b_n11, W_n21, b_n21, W_n22, b_n22, W_e11, b_e11, W_e12, b_e12, W_dec, b_dec):
    raise NotImplementedError("write your pallas kernel here")


# speedup vs baseline: 2.6122x; 2.6122x over previous
import functools

import jax
import jax.numpy as jnp
from jax import lax
from jax.experimental import pallas as pl
from jax.experimental.pallas import tpu as pltpu
from jax.experimental.pallas import tpu_sc as plsc

N = 10000
E = 320000
F = 128
IDX_ROWS = E // 128
NCHUNK = E // 256

_HI = jax.lax.Precision.HIGHEST

_mesh = plsc.VectorSubcoreMesh(core_axis_name="c", subcore_axis_name="s")


def _dot(a, b):
    return jnp.dot(a, b, preferred_element_type=jnp.float32, precision=_HI)


def _node_first_body(x_ref, W11, b11, W21, b21, W22, b22, Wa, Wb, be,
                     A_ref, B_ref):
    x = x_ref[...]
    h = jnp.maximum(_dot(x, W11[...]) + b11[...], 0.0)
    h = jnp.maximum(_dot(h, W21[...]) + b21[...], 0.0)
    h = jnp.maximum(_dot(h, W22[...]) + b22[...], 0.0)
    A_ref[...] = _dot(h, Wa[...]) + be[...]
    B_ref[...] = _dot(h, Wb[...])


def _node_rest_body(x_ref, W21, b21, W22, b22, Wa, Wb, be, A_ref, B_ref):
    x = x_ref[...]
    h = jnp.maximum(_dot(x, W21[...]) + b21[...], 0.0)
    h = jnp.maximum(_dot(h, W22[...]) + b22[...], 0.0)
    A_ref[...] = _dot(h, Wa[...]) + be[...]
    B_ref[...] = _dot(h, Wb[...])


def _full(shape):
    return pl.BlockSpec(shape, lambda i: (0, 0))


_NB = 2000


def _node_first(x, W11, b11, W21, b21, W22, b22, Wa, Wb, be):
    return pl.pallas_call(
        _node_first_body,
        grid=(N // _NB,),
        in_specs=[
            pl.BlockSpec((_NB, 128), lambda i: (i, 0)),
            _full((128, 256)), _full((1, 256)),
            _full((256, 128)), _full((1, 128)),
            _full((128, 128)), _full((1, 128)),
            _full((128, 128)), _full((128, 128)), _full((1, 128)),
        ],
        out_specs=[
            pl.BlockSpec((_NB, 128), lambda i: (i, 0)),
            pl.BlockSpec((_NB, 128), lambda i: (i, 0)),
        ],
        out_shape=[
            jax.ShapeDtypeStruct((N, 128), jnp.float32),
            jax.ShapeDtypeStruct((N, 128), jnp.float32),
        ],
    )(x, W11, b11, W21, b21, W22, b22, Wa, Wb, be)


def _node_rest(x, W21, b21, W22, b22, Wa, Wb, be):
    return pl.pallas_call(
        _node_rest_body,
        grid=(N // _NB,),
        in_specs=[
            pl.BlockSpec((_NB, 256), lambda i: (i, 0)),
            _full((256, 128)), _full((1, 128)),
            _full((128, 128)), _full((1, 128)),
            _full((128, 128)), _full((128, 128)), _full((1, 128)),
        ],
        out_specs=[
            pl.BlockSpec((_NB, 128), lambda i: (i, 0)),
            pl.BlockSpec((_NB, 128), lambda i: (i, 0)),
        ],
        out_shape=[
            jax.ShapeDtypeStruct((N, 128), jnp.float32),
            jax.ShapeDtypeStruct((N, 128), jnp.float32),
        ],
    )(x, W21, b21, W22, b22, Wa, Wb, be)


_EB = 2560


def _emm_body(e_ref, W, b, o_ref):
    o_ref[...] = jnp.maximum(_dot(e_ref[...], W[...]) + b[...], 0.0)


def _emm_final_body(e_ref, W, b, wd, bd, o_ref, p_ref):
    y = jnp.maximum(_dot(e_ref[...], W[...]) + b[...], 0.0)
    o_ref[...] = y
    logit = jnp.sum(y * wd[...], axis=1, keepdims=True) + bd[...]
    p_ref[...] = jax.nn.sigmoid(logit)


def _edge_mm(e, W, b):
    return pl.pallas_call(
        _emm_body,
        grid=(E // _EB,),
        in_specs=[
            pl.BlockSpec((_EB, 128), lambda i: (i, 0)),
            _full((128, 128)), _full((1, 128)),
        ],
        out_specs=pl.BlockSpec((_EB, 128), lambda i: (i, 0)),
        out_shape=jax.ShapeDtypeStruct((E, 128), jnp.float32),
    )(e, W, b)


def _edge_mm_final(e, W, b, wd, bd):
    return pl.pallas_call(
        _emm_final_body,
        grid=(E // _EB,),
        in_specs=[
            pl.BlockSpec((_EB, 128), lambda i: (i, 0)),
            _full((128, 128)), _full((1, 128)),
            _full((1, 128)), _full((1, 1)),
        ],
        out_specs=[
            pl.BlockSpec((_EB, 128), lambda i: (i, 0)),
            pl.BlockSpec((_EB, 1), lambda i: (i, 0)),
        ],
        out_shape=[
            jax.ShapeDtypeStruct((E, 128), jnp.float32),
            jax.ShapeDtypeStruct((E, 1), jnp.float32),
        ],
    )(e, W, b, wd, bd)


def _sc_gather(A, B, ii2, io2):
    @functools.partial(
        pl.kernel,
        out_type=jax.ShapeDtypeStruct((E, 128), jnp.float32),
        mesh=_mesh,
        scratch_types=[
            pltpu.VMEM((128, 128), jnp.float32),
            pltpu.VMEM((128, 128), jnp.float32),
            pltpu.SemaphoreType.DMA,
            pltpu.SemaphoreType.DMA,
        ],
    )
    def k(A_hbm, B_hbm, ii_hbm, io_hbm, e_hbm, tmpA, tmpB, semA, semB):
        def body(ii_v, io_v, o_v):
            cpA = pltpu.async_copy(A_hbm.at[ii_v.at[0, 0]], tmpA, semA)
            cpB = pltpu.async_copy(B_hbm.at[io_v.at[0, 0]], tmpB, semB)
            cpA.wait()
            cpB.wait()

            @pl.loop(0, 128)
            def _(r):
                for j in range(8):
                    sl = pl.ds(j * 16, 16)
                    o_v[r, sl] = jnp.maximum(tmpA[r, sl] + tmpB[r, sl], 0.0)

        pltpu.emit_pipeline(
            body,
            grid=(IDX_ROWS,),
            in_specs=[
                pl.BlockSpec((1, 1, 128), lambda i: (i, 0, 0)),
                pl.BlockSpec((1, 1, 128), lambda i: (i, 0, 0)),
            ],
            out_specs=[pl.BlockSpec((128, 128), lambda i: (i, 0))],
            core_axis_name=("c", "s"),
            dimension_semantics=(pltpu.PARALLEL,),
        )(ii_hbm, io_hbm, e_hbm)

    return k(A, B, ii2.reshape(IDX_ROWS, 1, 128), io2.reshape(IDX_ROWS, 1, 128))


def _sc_scatter(f, ii2, io2):
    @functools.partial(
        pl.kernel,
        out_type=jax.ShapeDtypeStruct((N, 256), jnp.float32),
        mesh=_mesh,
        scratch_types=[
            pltpu.VMEM_SHARED((N, 128), jnp.float32),
            pltpu.VMEM((2, 128, 128), jnp.float32),
            pltpu.VMEM((2, 1, 128), jnp.int32),
            pltpu.VMEM((80, 128), jnp.float32),
            pltpu.SemaphoreType.DMA((2,)),
            pltpu.SemaphoreType.DMA((2,)),
            pltpu.SemaphoreType.DMA((2,)),
        ],
    )
    def k(f_hbm, ii_hbm, io_hbm, S_hbm, acc, ebuf, ibuf, zbuf,
          esems, isems, scsems):
        c = lax.axis_index("c")
        s = lax.axis_index("s")

        zero = jnp.zeros((16,), jnp.float32)

        @pl.loop(0, 80)
        def _(r):
            for j in range(8):
                zbuf[r, pl.ds(j * 16, 16)] = zero

        nz = jnp.where(s < 15, 8, 5)

        @pl.loop(0, nz)
        def _(q):
            pltpu.sync_copy(zbuf, acc.at[pl.ds(s * 640 + q * 80, 80)])

        plsc.subcore_barrier()

        def edge_loop(idx_hbm):
            def issue_loads(t, p):
                pltpu.async_copy(f_hbm.at[pl.ds(t * 128, 128)], ebuf.at[p],
                                 esems.at[p])
                pltpu.async_copy(idx_hbm.at[t], ibuf.at[p], isems.at[p])

            def wait_loads(t, p):
                pltpu.make_async_copy(f_hbm.at[pl.ds(t * 128, 128)],
                                      ebuf.at[p], esems.at[p]).wait()
                pltpu.make_async_copy(idx_hbm.at[t], ibuf.at[p],
                                      isems.at[p]).wait()

            def issue_scatters(p):
                pltpu.async_copy(ebuf.at[p], acc.at[ibuf.at[p].at[0]],
                                 scsems.at[p], add=True)

            def wait_scatters(p):
                pltpu.make_async_copy(ebuf.at[p], acc.at[ibuf.at[p].at[0]],
                                      scsems.at[p]).wait()

            issue_loads(s, 0)

            def step(kk, p, q):
                t = s + 16 * kk

                @pl.when(t < IDX_ROWS)
                def _():
                    wait_loads(t, p)
                    issue_scatters(p)

                @pl.when((kk >= 1) & (t - 16 < IDX_ROWS))
                def _():
                    wait_scatters(q)

                @pl.when(t + 16 < IDX_ROWS)
                def _():
                    issue_loads(t + 16, q)

            @pl.loop(0, 158, step=2)
            def _(kk):
                step(kk, 0, 1)
                step(kk + 1, 1, 0)

        @pl.when(c == 0)
        def _():
            edge_loop(ii_hbm)

        @pl.when(c == 1)
        def _():
            edge_loop(io_hbm)

        plsc.subcore_barrier()

        col = pl.ds(0, 128)

        def writeout(rows):
            @pl.when(c == 0)
            def _():
                pltpu.sync_copy(acc.at[rows], S_hbm.at[rows, pl.ds(0, 128)])

            @pl.when(c == 1)
            def _():
                pltpu.sync_copy(acc.at[rows], S_hbm.at[rows, pl.ds(128, 128)])

        @pl.when(s < 15)
        def _():
            writeout(pl.ds(s * 640, 640))

        @pl.when(s == 15)
        def _():
            writeout(pl.ds(9600, 400))

    return k(f, ii2.reshape(IDX_ROWS, 1, 128), io2.reshape(IDX_ROWS, 1, 128))


def kernel(inputs, n2e_in, n2e_out, x_idx, y_idx,
           W_n11, b_n11, W_n21, b_n21, W_n22, b_n22,
           W_e11, b_e11, W_e12, b_e12, W_dec, b_dec):
    Wa = W_e11[:128]
    Wb = W_e11[128:]
    ii2 = n2e_in.reshape(IDX_ROWS, 128)
    io2 = n2e_out.reshape(IDX_ROWS, 128)
    b11 = b_n11.reshape(1, 256)
    b21 = b_n21.reshape(1, 128)
    b22 = b_n22.reshape(1, 128)
    be = b_e11.reshape(1, 128)
    b12 = b_e12.reshape(1, 128)
    wd = W_dec.reshape(1, 128)
    bd = b_dec.reshape(1, 1)

    A, B = _node_first(inputs, W_n11, b11, W_n21, b21, W_n22, b22, Wa, Wb, be)
    e = _sc_gather(A, B, ii2, io2)
    f = _edge_mm(e, W_e12, b12)
    S = _sc_scatter(f, ii2, io2)
    A, B = _node_rest(S, W_n21, b21, W_n22, b22, Wa, Wb, be)
    e = _sc_gather(A, B, ii2, io2)
    xedge, p = _edge_mm_final(e, W_e12, b12, wd, bd)
    return p.reshape(-1), xedge


# speedup vs baseline: 4.5267x; 1.7329x over previous
import dataclasses
import functools

import jax
import jax.numpy as jnp
from jax import lax
from jax.experimental import pallas as pl
from jax.experimental.pallas import tpu as pltpu
from jax.experimental.pallas import tpu_sc as plsc

N = 10000
E = 320000
F = 128
IDX_ROWS = E // 128
NCHUNK = E // 256

_HI = jax.lax.Precision.HIGHEST

_mesh = plsc.VectorSubcoreMesh(core_axis_name="c", subcore_axis_name="s")

_sc_params = pltpu.CompilerParams()
if "needs_layout_passes" in pltpu.CompilerParams.__dataclass_fields__:
    _sc_params = dataclasses.replace(_sc_params, needs_layout_passes=False)


def _dot(a, b):
    return jnp.dot(a, b, preferred_element_type=jnp.float32, precision=_HI)


def _node_first_body(x_ref, W11, b11, W21, b21, W22, b22, Wa, Wb, be,
                     A_ref, B_ref):
    x = x_ref[...]
    h = jnp.maximum(_dot(x, W11[...]) + b11[...], 0.0)
    h = jnp.maximum(_dot(h, W21[...]) + b21[...], 0.0)
    h = jnp.maximum(_dot(h, W22[...]) + b22[...], 0.0)
    A_ref[...] = _dot(h, Wa[...]) + be[...]
    B_ref[...] = _dot(h, Wb[...])


def _node_rest_body(x_ref, W21, b21, W22, b22, Wa, Wb, be, A_ref, B_ref):
    x = x_ref[...]
    h = jnp.maximum(_dot(x, W21[...]) + b21[...], 0.0)
    h = jnp.maximum(_dot(h, W22[...]) + b22[...], 0.0)
    A_ref[...] = _dot(h, Wa[...]) + be[...]
    B_ref[...] = _dot(h, Wb[...])


def _full(shape):
    return pl.BlockSpec(shape, lambda i: (0, 0))


_NB = 2000


def _node_first(x, W11, b11, W21, b21, W22, b22, Wa, Wb, be):
    return pl.pallas_call(
        _node_first_body,
        grid=(N // _NB,),
        in_specs=[
            pl.BlockSpec((_NB, 128), lambda i: (i, 0)),
            _full((128, 256)), _full((1, 256)),
            _full((256, 128)), _full((1, 128)),
            _full((128, 128)), _full((1, 128)),
            _full((128, 128)), _full((128, 128)), _full((1, 128)),
        ],
        out_specs=[
            pl.BlockSpec((_NB, 128), lambda i: (i, 0)),
            pl.BlockSpec((_NB, 128), lambda i: (i, 0)),
        ],
        out_shape=[
            jax.ShapeDtypeStruct((N, 128), jnp.float32),
            jax.ShapeDtypeStruct((N, 128), jnp.float32),
        ],
    )(x, W11, b11, W21, b21, W22, b22, Wa, Wb, be)


def _node_rest(x, W21, b21, W22, b22, Wa, Wb, be):
    return pl.pallas_call(
        _node_rest_body,
        grid=(N // _NB,),
        in_specs=[
            pl.BlockSpec((_NB, 256), lambda i: (i, 0)),
            _full((256, 128)), _full((1, 128)),
            _full((128, 128)), _full((1, 128)),
            _full((128, 128)), _full((128, 128)), _full((1, 128)),
        ],
        out_specs=[
            pl.BlockSpec((_NB, 128), lambda i: (i, 0)),
            pl.BlockSpec((_NB, 128), lambda i: (i, 0)),
        ],
        out_shape=[
            jax.ShapeDtypeStruct((N, 128), jnp.float32),
            jax.ShapeDtypeStruct((N, 128), jnp.float32),
        ],
    )(x, W21, b21, W22, b22, Wa, Wb, be)


_EB = 2560


def _emm_body(e_ref, W, b, o_ref):
    x = e_ref[...].astype(jnp.float32)
    o_ref[...] = jnp.maximum(_dot(x, W[...]) + b[...], 0.0)


def _emm_final_body(e_ref, W, b, wd, bd, o_ref, p_ref):
    x = e_ref[...].astype(jnp.float32)
    y = jnp.maximum(_dot(x, W[...]) + b[...], 0.0)
    o_ref[...] = y
    logit = jnp.sum(y * wd[...], axis=1, keepdims=True) + bd[...]
    p_ref[...] = jax.nn.sigmoid(logit)


def _edge_mm(e, W, b):
    return pl.pallas_call(
        _emm_body,
        grid=(E // _EB,),
        in_specs=[
            pl.BlockSpec((_EB, 128), lambda i: (i, 0)),
            _full((128, 128)), _full((1, 128)),
        ],
        out_specs=pl.BlockSpec((_EB, 128), lambda i: (i, 0)),
        out_shape=jax.ShapeDtypeStruct((E, 128), jnp.float32),
    )(e, W, b)


def _edge_mm_final(e, W, b, wd, bd):
    return pl.pallas_call(
        _emm_final_body,
        grid=(E // _EB,),
        in_specs=[
            pl.BlockSpec((_EB, 128), lambda i: (i, 0)),
            _full((128, 128)), _full((1, 128)),
            _full((1, 128)), _full((1, 1)),
        ],
        out_specs=[
            pl.BlockSpec((_EB, 128), lambda i: (i, 0)),
            pl.BlockSpec((_EB, 1), lambda i: (i, 0)),
        ],
        out_shape=[
            jax.ShapeDtypeStruct((E, 128), jnp.float32),
            jax.ShapeDtypeStruct((E, 1), jnp.float32),
        ],
    )(e, W, b, wd, bd)


def _sc_gather(A, B, ii2, io2):
    @functools.partial(
        pl.kernel,
        out_type=jax.ShapeDtypeStruct((E, 128), jnp.float32),
        mesh=_mesh,
        compiler_params=_sc_params,
        scratch_types=[
            pltpu.VMEM((2, 128, 128), jnp.float32),
            pltpu.VMEM((2, 128, 128), jnp.float32),
            pltpu.VMEM((2, 128, 128), jnp.float32),
            pltpu.VMEM((2, 1, 128), jnp.int32),
            pltpu.VMEM((2, 1, 128), jnp.int32),
            pltpu.SemaphoreType.DMA((2,)),
            pltpu.SemaphoreType.DMA((2,)),
            pltpu.SemaphoreType.DMA((2,)),
            pltpu.SemaphoreType.DMA((2,)),
            pltpu.SemaphoreType.DMA((2,)),
        ],
    )
    def k(A_hbm, B_hbm, ii_hbm, io_hbm, e_hbm, gbufA, gbufB, obuf,
          iibuf, iobuf, gsA, gsB, isI, isO, wsems):
        c = lax.axis_index("c")
        s = lax.axis_index("s")
        wid = s * 2 + c

        def tix(kk):
            return wid + 32 * kk

        def issue_idx(kk, p):
            t = tix(kk)
            pltpu.async_copy(ii_hbm.at[t], iibuf.at[p], isI.at[p])
            pltpu.async_copy(io_hbm.at[t], iobuf.at[p], isO.at[p])

        def wait_idx(kk, p):
            t = tix(kk)
            pltpu.make_async_copy(ii_hbm.at[t], iibuf.at[p],
                                  isI.at[p]).wait()
            pltpu.make_async_copy(io_hbm.at[t], iobuf.at[p],
                                  isO.at[p]).wait()

        def issue_gathers(p):
            pltpu.async_copy(A_hbm.at[iibuf.at[p].at[0]], gbufA.at[p],
                             gsA.at[p])
            pltpu.async_copy(B_hbm.at[iobuf.at[p].at[0]], gbufB.at[p],
                             gsB.at[p])

        def wait_gathers(p):
            pltpu.make_async_copy(A_hbm.at[iibuf.at[p].at[0]], gbufA.at[p],
                                  gsA.at[p]).wait()
            pltpu.make_async_copy(B_hbm.at[iobuf.at[p].at[0]], gbufB.at[p],
                                  gsB.at[p]).wait()

        def wait_write(p):
            pltpu.make_async_copy(obuf.at[p], e_hbm.at[pl.ds(0, 128)],
                                  wsems.at[p]).wait()

        issue_idx(0, 0)
        wait_idx(0, 0)
        issue_gathers(0)
        issue_idx(1, 1)

        def step(kk, p, q):
            t = tix(kk)

            @pl.when(t < IDX_ROWS)
            def _():
                wait_gathers(p)

            @pl.when(t + 32 < IDX_ROWS)
            def _():
                wait_idx(kk + 1, q)
                issue_gathers(q)

            @pl.when(t + 64 < IDX_ROWS)
            def _():
                issue_idx(kk + 2, p)

            @pl.when(t < IDX_ROWS)
            def _():
                @pl.when(kk >= 2)
                def _():
                    wait_write(p)

                @pl.loop(0, 128)
                def _(r):
                    for j in range(8):
                        sl = pl.ds(j * 16, 16)
                        obuf[p, r, sl] = jnp.maximum(
                            gbufA[p, r, sl] + gbufB[p, r, sl], 0.0)

                pltpu.async_copy(obuf.at[p],
                                 e_hbm.at[pl.ds(t * 128, 128)],
                                 wsems.at[p])

        @pl.loop(0, 80, step=2)
        def _(kk):
            step(kk, 0, 1)
            step(kk + 1, 1, 0)

        wait_write(0)
        wait_write(1)

    return k(A, B, ii2.reshape(IDX_ROWS, 1, 128),
             io2.reshape(IDX_ROWS, 1, 128))


def _sc_scatter(f, ii2, io2):
    @functools.partial(
        pl.kernel,
        out_type=jax.ShapeDtypeStruct((N, 256), jnp.float32),
        mesh=_mesh,
        scratch_types=[
            pltpu.VMEM_SHARED((N, 128), jnp.float32),
            pltpu.VMEM((2, 128, 128), jnp.float32),
            pltpu.VMEM((2, 1, 128), jnp.int32),
            pltpu.VMEM((80, 128), jnp.float32),
            pltpu.SemaphoreType.DMA((2,)),
            pltpu.SemaphoreType.DMA((2,)),
            pltpu.SemaphoreType.DMA((2,)),
        ],
    )
    def k(f_hbm, ii_hbm, io_hbm, S_hbm, acc, ebuf, ibuf, zbuf,
          esems, isems, scsems):
        c = lax.axis_index("c")
        s = lax.axis_index("s")

        zero = jnp.zeros((16,), jnp.float32)

        @pl.loop(0, 80)
        def _(r):
            for j in range(8):
                zbuf[r, pl.ds(j * 16, 16)] = zero

        nz = jnp.where(s < 15, 8, 5)

        @pl.loop(0, nz)
        def _(q):
            pltpu.sync_copy(zbuf, acc.at[pl.ds(s * 640 + q * 80, 80)])

        plsc.subcore_barrier()

        def edge_loop(idx_hbm):
            def issue_loads(t, p):
                pltpu.async_copy(f_hbm.at[pl.ds(t * 128, 128)], ebuf.at[p],
                                 esems.at[p])
                pltpu.async_copy(idx_hbm.at[t], ibuf.at[p], isems.at[p])

            def wait_loads(t, p):
                pltpu.make_async_copy(f_hbm.at[pl.ds(t * 128, 128)],
                                      ebuf.at[p], esems.at[p]).wait()
                pltpu.make_async_copy(idx_hbm.at[t], ibuf.at[p],
                                      isems.at[p]).wait()

            def issue_scatters(p):
                pltpu.async_copy(ebuf.at[p], acc.at[ibuf.at[p].at[0]],
                                 scsems.at[p], add=True)

            def wait_scatters(p):
                pltpu.make_async_copy(ebuf.at[p], acc.at[ibuf.at[p].at[0]],
                                      scsems.at[p]).wait()

            issue_loads(s, 0)

            def step(kk, p, q):
                t = s + 16 * kk

                @pl.when(t < IDX_ROWS)
                def _():
                    wait_loads(t, p)
                    issue_scatters(p)

                @pl.when((kk >= 1) & (t - 16 < IDX_ROWS))
                def _():
                    wait_scatters(q)

                @pl.when(t + 16 < IDX_ROWS)
                def _():
                    issue_loads(t + 16, q)

            @pl.loop(0, 158, step=2)
            def _(kk):
                step(kk, 0, 1)
                step(kk + 1, 1, 0)

        @pl.when(c == 0)
        def _():
            edge_loop(ii_hbm)

        @pl.when(c == 1)
        def _():
            edge_loop(io_hbm)

        plsc.subcore_barrier()

        col = pl.ds(0, 128)

        def writeout(rows):
            @pl.when(c == 0)
            def _():
                pltpu.sync_copy(acc.at[rows], S_hbm.at[rows, pl.ds(0, 128)])

            @pl.when(c == 1)
            def _():
                pltpu.sync_copy(acc.at[rows], S_hbm.at[rows, pl.ds(128, 128)])

        @pl.when(s < 15)
        def _():
            writeout(pl.ds(s * 640, 640))

        @pl.when(s == 15)
        def _():
            writeout(pl.ds(9600, 400))

    return k(f, ii2.reshape(IDX_ROWS, 1, 128), io2.reshape(IDX_ROWS, 1, 128))


def kernel(inputs, n2e_in, n2e_out, x_idx, y_idx,
           W_n11, b_n11, W_n21, b_n21, W_n22, b_n22,
           W_e11, b_e11, W_e12, b_e12, W_dec, b_dec):
    Wa = W_e11[:128]
    Wb = W_e11[128:]
    ii2 = n2e_in.reshape(IDX_ROWS, 128)
    io2 = n2e_out.reshape(IDX_ROWS, 128)
    b11 = b_n11.reshape(1, 256)
    b21 = b_n21.reshape(1, 128)
    b22 = b_n22.reshape(1, 128)
    be = b_e11.reshape(1, 128)
    b12 = b_e12.reshape(1, 128)
    wd = W_dec.reshape(1, 128)
    bd = b_dec.reshape(1, 1)

    A, B = _node_first(inputs, W_n11, b11, W_n21, b21, W_n22, b22, Wa, Wb, be)
    e = _sc_gather(A, B, ii2, io2)
    f = _edge_mm(e, W_e12, b12)
    S = _sc_scatter(f, ii2, io2)
    A, B = _node_rest(S, W_n21, b21, W_n22, b22, Wa, Wb, be)
    e = _sc_gather(A, B, ii2, io2)
    xedge, p = _edge_mm_final(e, W_e12, b12, wd, bd)
    return p.reshape(-1), xedge
